# Initial kernel scaffold; baseline (speedup 1.0000x reference)
import jax
import jax.numpy as jnp
from jax.experimental import pallas as pl


def kernel(x, tkn_emb_table):
    raise NotImplementedError("write your pallas kernel here")


# speedup vs baseline: 1.0261x; 1.0261x over previous
import functools

import jax
import jax.numpy as jnp
from jax import lax
from jax.experimental import pallas as pl
from jax.experimental.pallas import tpu as pltpu
from jax.experimental.pallas import tpu_sc as plsc

VOCAB = 1000
B = 1024
S = 50
N = B * S
NC = 2
NS = 16
NW = NC * NS
PER_W = N // NW
K = 16
NG = PER_W // K


def _sc_gather(idx, table_flat):
  mesh = plsc.VectorSubcoreMesh(core_axis_name="c", subcore_axis_name="s")

  @functools.partial(
      pl.kernel,
      mesh=mesh,
      out_type=jax.ShapeDtypeStruct((N * VOCAB,), jnp.float32),
      scratch_types=[
          pltpu.VMEM((PER_W,), jnp.int32),
          pltpu.VMEM((K * VOCAB,), jnp.float32),
          pltpu.VMEM((K * VOCAB,), jnp.float32),
          pltpu.SemaphoreType.DMA,
          pltpu.SemaphoreType.DMA,
          pltpu.SemaphoreType.DMA,
          pltpu.SemaphoreType.DMA,
      ],
  )
  def k(idx_hbm, table_hbm, out_hbm, idx_v, buf0, buf1, gsem0, gsem1,
        ssem0, ssem1):
    wid = lax.axis_index("s") * NC + lax.axis_index("c")
    base = wid * PER_W
    pltpu.sync_copy(idx_hbm.at[pl.ds(base, PER_W)], idx_v)

    bufs = (buf0, buf1)
    gsems = (gsem0, gsem1)
    ssems = (ssem0, ssem1)

    def fire_gathers(g, p):
      iv = idx_v[pl.ds(pl.multiple_of(g * K, K), K)] * VOCAB
      for j in range(K):
        src = pl.multiple_of(iv[j], 8)
        pltpu.async_copy(table_hbm.at[pl.ds(src, VOCAB)],
                         bufs[p].at[pl.ds(j * VOCAB, VOCAB)], gsems[p])

    def out_slice(g):
      off = pl.multiple_of((base + g * K) * VOCAB, 8)
      return out_hbm.at[pl.ds(off, K * VOCAB)]

    fire_gathers(0, 0)
    fire_gathers(1, 1)

    @pl.loop(0, NG, step=2)
    def body(gg):
      for p in range(2):
        g = gg + p
        pltpu.make_async_copy(out_slice(g), bufs[p], gsems[p]).wait()

        @pl.loop(0, K)
        def _(j):
          r = base + g * K + j
          dst = pl.multiple_of(r * VOCAB, 8)
          pltpu.async_copy(bufs[p].at[pl.ds(j * VOCAB, VOCAB)],
                           out_hbm.at[pl.ds(dst, VOCAB)], ssems[p])

        pltpu.make_async_copy(bufs[p], out_slice(g), ssems[p]).wait()

        @pl.when(g + 2 < NG)
        def _():
          fire_gathers(g + 2, p)

  return k(idx, table_flat)


def kernel(x, tkn_emb_table):
  idx = x.reshape(-1).astype(jnp.int32)
  out = _sc_gather(idx, tkn_emb_table.reshape(-1))
  return out.reshape(B, S, VOCAB)


# speedup vs baseline: 1.0351x; 1.0088x over previous
import functools

import jax
import jax.numpy as jnp
from jax import lax
from jax.experimental import pallas as pl
from jax.experimental.pallas import tpu as pltpu
from jax.experimental.pallas import tpu_sc as plsc

VOCAB = 1000
B = 1024
S = 50
N = B * S
NC = 2
NS = 16
NW = NC * NS
PER_W = N // NW
K = 32
NG = PER_W // K


def _sc_gather(idx, table_flat):
  mesh = plsc.VectorSubcoreMesh(core_axis_name="c", subcore_axis_name="s")

  @functools.partial(
      pl.kernel,
      mesh=mesh,
      out_type=jax.ShapeDtypeStruct((N * VOCAB,), jnp.float32),
      scratch_types=[
          pltpu.VMEM((PER_W,), jnp.int32),
          pltpu.VMEM((K * VOCAB,), jnp.float32),
          pltpu.VMEM((K * VOCAB,), jnp.float32),
          pltpu.SemaphoreType.DMA,
          pltpu.SemaphoreType.DMA,
      ],
  )
  def k(idx_hbm, table_hbm, out_hbm, idx_v, buf0, buf1, gsem0, gsem1):
    wid = lax.axis_index("s") * NC + lax.axis_index("c")
    base = wid * PER_W
    pltpu.sync_copy(idx_hbm.at[pl.ds(base, PER_W)], idx_v)

    bufs = (buf0, buf1)
    gsems = (gsem0, gsem1)

    def fire_gathers(g, p):
      for h in range(K // 16):
        iv = idx_v[pl.ds(pl.multiple_of(g * K + h * 16, 16), 16)] * VOCAB
        for t in range(16):
          src = pl.multiple_of(iv[t], 8)
          pltpu.async_copy(table_hbm.at[pl.ds(src, VOCAB)],
                           bufs[p].at[pl.ds((h * 16 + t) * VOCAB, VOCAB)],
                           gsems[p])

    def out_slice(g):
      off = pl.multiple_of((base + g * K) * VOCAB, 8)
      return out_hbm.at[pl.ds(off, K * VOCAB)]

    fire_gathers(0, 0)
    fire_gathers(1, 1)

    @pl.loop(0, NG, step=2)
    def body(gg):
      for p in range(2):
        g = gg + p
        pltpu.make_async_copy(out_slice(g), bufs[p], gsems[p]).wait()

        pltpu.sync_copy(bufs[p], out_slice(g))

        @pl.when(g + 2 < NG)
        def _():
          fire_gathers(g + 2, p)

  return k(idx, table_flat)


def kernel(x, tkn_emb_table):
  idx = x.reshape(-1).astype(jnp.int32)
  out = _sc_gather(idx, tkn_emb_table.reshape(-1))
  return out.reshape(B, S, VOCAB)
